# Initial kernel scaffold; baseline (speedup 1.0000x reference)
import functools

import jax
import jax.numpy as jnp
from jax import lax
from jax.experimental import pallas as pl
from jax.experimental.pallas import tpu as pltpu
from jax.experimental.pallas import tpu_sc as plsc

N_NODES = 10000
N_EDGES = 320000
D = 128
N_GRAPHS = 64

_NC = 2
_NS = 16
_NW = _NC * _NS
_CHUNK = 128
_CH = 79
_E_PAD = _NW * _CH * _CHUNK
_RPT = 640
_N_PAD = _NS * _RPT


@functools.partial(
    pl.kernel,
    mesh=plsc.VectorSubcoreMesh(core_axis_name="c", subcore_axis_name="s"),
    out_type=jax.ShapeDtypeStruct((_NC, _N_PAD, D), jnp.float32),
    scratch_types=[
        pltpu.VMEM((_CH, _CHUNK), jnp.int32),
        pltpu.VMEM((_CH, _CHUNK), jnp.int32),
        pltpu.VMEM((_CHUNK, D), jnp.float32),
        pltpu.VMEM_SHARED((_N_PAD, D), jnp.float32),
        pltpu.SemaphoreType.DMA,
    ],
)
def _sc_agg(x_hbm, srcs_hbm, dsts_hbm, zeros_hbm, out_hbm,
            src_v, dst_v, rows_v, agg_sh, sem):
    cid = lax.axis_index("c")
    sid = lax.axis_index("s")
    wid = cid * _NS + sid
    pltpu.sync_copy(srcs_hbm.at[wid], src_v)
    pltpu.sync_copy(dsts_hbm.at[wid], dst_v)
    pltpu.sync_copy(zeros_hbm, agg_sh.at[pl.ds(sid * _RPT, _RPT)])
    plsc.subcore_barrier()

    def body(i, carry):
        pltpu.async_copy(x_hbm.at[src_v.at[i]], rows_v, sem).wait()
        pltpu.sync_copy(rows_v, agg_sh.at[dst_v.at[i]], add=True)
        return carry

    lax.fori_loop(0, _CH, body, 0)
    plsc.subcore_barrier()
    pltpu.sync_copy(agg_sh.at[pl.ds(sid * _RPT, _RPT)],
                    out_hbm.at[cid, pl.ds(sid * _RPT, _RPT)])


_PREC = lax.Precision.HIGHEST


def _mlp_body(x_ref, p0_ref, p1_ref, wa_ref, ba_ref, wb_ref, bb_ref, o_ref):
    agg = x_ref[...] + p0_ref[0] + p1_ref[0]
    h = jnp.dot(agg, wa_ref[...], precision=_PREC,
                preferred_element_type=jnp.float32) + ba_ref[...]
    h = jnp.maximum(h, 0.0)
    o = jnp.dot(h, wb_ref[...], precision=_PREC,
                preferred_element_type=jnp.float32) + bb_ref[...]
    o_ref[...] = jnp.maximum(o, 0.0)


def _mlp(x, p, wa, ba, wb, bb):
    blk = 1000
    return pl.pallas_call(
        _mlp_body,
        grid=(N_NODES // blk,),
        in_specs=[
            pl.BlockSpec((blk, D), lambda i: (i, 0)),
            pl.BlockSpec((1, blk, D), lambda i: (0, i, 0)),
            pl.BlockSpec((1, blk, D), lambda i: (1, i, 0)),
            pl.BlockSpec((D, D), lambda i: (0, 0)),
            pl.BlockSpec((1, D), lambda i: (0, 0)),
            pl.BlockSpec((D, D), lambda i: (0, 0)),
            pl.BlockSpec((1, D), lambda i: (0, 0)),
        ],
        out_specs=pl.BlockSpec((blk, D), lambda i: (i, 0)),
        out_shape=jax.ShapeDtypeStruct((N_NODES, D), jnp.float32),
    )(x, p, p, wa, ba, wb, bb)


def _pool_body(h_ref, b_ref, wf1_ref, bf1_ref, wf2_ref, bf2_ref, o_ref):
    h = h_ref[...]
    seg = b_ref[...]
    gids = lax.broadcasted_iota(jnp.int32, (1, N_GRAPHS), 1)
    onehot = (seg == gids).astype(jnp.float32)
    sums = lax.dot_general(onehot, h, (((0,), (0,)), ((), ())),
                           precision=_PREC,
                           preferred_element_type=jnp.float32)
    counts = jnp.sum(onehot, axis=0)[:, None]
    mean = sums / jnp.maximum(counts, 1.0)

    def mbody(g, acc):
        mg = jnp.max(jnp.where(seg == g, h, -jnp.inf), axis=0)
        return lax.dynamic_update_slice(acc, mg[None], (g, 0))

    maxes = lax.fori_loop(0, N_GRAPHS, mbody,
                          jnp.full((N_GRAPHS, D), -jnp.inf, jnp.float32))
    maxes = jnp.where(counts > 0.0, maxes, 0.0)
    pooled = jnp.concatenate([mean, maxes], axis=1)
    z = jnp.dot(pooled, wf1_ref[...], precision=_PREC,
                preferred_element_type=jnp.float32) + bf1_ref[...]
    z = jnp.maximum(z, 0.0)
    o_ref[...] = jnp.dot(z, wf2_ref[...], precision=_PREC,
                         preferred_element_type=jnp.float32) + bf2_ref[...]


def _pool(h, seg, wf1, bf1, wf2, bf2):
    return pl.pallas_call(
        _pool_body,
        out_shape=jax.ShapeDtypeStruct((N_GRAPHS, 1), jnp.float32),
    )(h, seg, wf1, bf1, wf2, bf2)


def kernel(x, edge_index, batch,
           W1a, b1a, W1b, b1b, W2a, b2a, W2b, b2b, Wf1, bf1, Wf2, bf2):
    src = edge_index[0].astype(jnp.int32)
    dst = edge_index[1].astype(jnp.int32)
    pad = _E_PAD - N_EDGES
    srcs = jnp.concatenate([src, jnp.zeros((pad,), jnp.int32)])
    srcs = srcs.reshape(_NW, _CH, _CHUNK)
    dsts = jnp.concatenate([dst, jnp.full((pad,), N_NODES, jnp.int32)])
    dsts = dsts.reshape(_NW, _CH, _CHUNK)
    zeros = jnp.zeros((_RPT, D), jnp.float32)

    p = _sc_agg(x, srcs, dsts, zeros)
    h1 = _mlp(x, p, W1a, b1a.reshape(1, D), W1b, b1b.reshape(1, D))
    q = _sc_agg(h1, srcs, dsts, zeros)
    h2 = _mlp(h1, q, W2a, b2a.reshape(1, D), W2b, b2b.reshape(1, D))
    return _pool(h2, batch.astype(jnp.int32).reshape(N_NODES, 1),
                 Wf1, bf1.reshape(1, D), Wf2, bf2.reshape(1, 1))


# speedup vs baseline: 3.9222x; 3.9222x over previous
import functools

import jax
import jax.numpy as jnp
from jax import lax
from jax.experimental import pallas as pl
from jax.experimental.pallas import tpu as pltpu
from jax.experimental.pallas import tpu_sc as plsc

N_NODES = 10000
N_EDGES = 320000
D = 128
N_GRAPHS = 64

_NC = 2
_NS = 16
_NW = _NC * _NS
_CHUNK = 128
_CH = 79
_E_PAD = _NW * _CH * _CHUNK
_RPT = 640
_N_PAD = _NS * _RPT


@functools.partial(
    pl.kernel,
    mesh=plsc.VectorSubcoreMesh(core_axis_name="c", subcore_axis_name="s"),
    out_type=jax.ShapeDtypeStruct((_NC, _N_PAD, D), jnp.float32),
    scratch_types=[
        pltpu.VMEM((_CH, _CHUNK), jnp.int32),
        pltpu.VMEM((_CH, _CHUNK), jnp.int32),
        pltpu.VMEM((_CHUNK, D), jnp.float32),
        pltpu.VMEM_SHARED((_N_PAD, D), jnp.float32),
        pltpu.SemaphoreType.DMA,
    ],
)
def _sc_agg(x_hbm, srcs_hbm, dsts_hbm, zeros_hbm, out_hbm,
            src_v, dst_v, rows_v, agg_sh, sem):
    cid = lax.axis_index("c")
    sid = lax.axis_index("s")
    wid = cid * _NS + sid
    pltpu.sync_copy(srcs_hbm.at[wid], src_v)
    pltpu.sync_copy(dsts_hbm.at[wid], dst_v)
    pltpu.sync_copy(zeros_hbm, agg_sh.at[pl.ds(sid * _RPT, _RPT)])
    plsc.subcore_barrier()

    def body(i, carry):
        pltpu.async_copy(x_hbm.at[src_v.at[i]], rows_v, sem).wait()
        pltpu.sync_copy(rows_v, agg_sh.at[dst_v.at[i]], add=True)
        return carry

    lax.fori_loop(0, _CH, body, 0)
    plsc.subcore_barrier()
    pltpu.sync_copy(agg_sh.at[pl.ds(sid * _RPT, _RPT)],
                    out_hbm.at[cid, pl.ds(sid * _RPT, _RPT)])


_PREC = lax.Precision.HIGHEST


def _mlp_body(x_ref, p0_ref, p1_ref, wa_ref, ba_ref, wb_ref, bb_ref, o_ref):
    agg = x_ref[...] + p0_ref[0] + p1_ref[0]
    h = jnp.dot(agg, wa_ref[...], precision=_PREC,
                preferred_element_type=jnp.float32) + ba_ref[...]
    h = jnp.maximum(h, 0.0)
    o = jnp.dot(h, wb_ref[...], precision=_PREC,
                preferred_element_type=jnp.float32) + bb_ref[...]
    o_ref[...] = jnp.maximum(o, 0.0)


def _mlp(x, p, wa, ba, wb, bb):
    blk = 1000
    return pl.pallas_call(
        _mlp_body,
        grid=(N_NODES // blk,),
        in_specs=[
            pl.BlockSpec((blk, D), lambda i: (i, 0)),
            pl.BlockSpec((1, blk, D), lambda i: (0, i, 0)),
            pl.BlockSpec((1, blk, D), lambda i: (1, i, 0)),
            pl.BlockSpec((D, D), lambda i: (0, 0)),
            pl.BlockSpec((1, D), lambda i: (0, 0)),
            pl.BlockSpec((D, D), lambda i: (0, 0)),
            pl.BlockSpec((1, D), lambda i: (0, 0)),
        ],
        out_specs=pl.BlockSpec((blk, D), lambda i: (i, 0)),
        out_shape=jax.ShapeDtypeStruct((N_NODES, D), jnp.float32),
    )(x, p, p, wa, ba, wb, bb)


def _pool_body(h_ref, b_ref, wf1_ref, bf1_ref, wf2_ref, bf2_ref, o_ref, max_ref):
    h = h_ref[...]
    seg = b_ref[...]
    gids = lax.broadcasted_iota(jnp.int32, (1, N_GRAPHS), 1)
    onehot = (seg == gids).astype(jnp.float32)
    sums = lax.dot_general(onehot, h, (((0,), (0,)), ((), ())),
                           precision=_PREC,
                           preferred_element_type=jnp.float32)
    counts = jnp.sum(onehot, axis=0)[:, None]
    mean = sums / jnp.maximum(counts, 1.0)

    def mbody(g, carry):
        mg = jnp.max(jnp.where(seg == g, h, -jnp.inf), axis=0)
        max_ref[pl.ds(g, 1), :] = mg[None]
        return carry

    lax.fori_loop(0, N_GRAPHS, mbody, 0)
    maxes = jnp.where(counts > 0.0, max_ref[...], 0.0)
    pooled = jnp.concatenate([mean, maxes], axis=1)
    z = jnp.dot(pooled, wf1_ref[...], precision=_PREC,
                preferred_element_type=jnp.float32) + bf1_ref[...]
    z = jnp.maximum(z, 0.0)
    o_ref[...] = jnp.dot(z, wf2_ref[...], precision=_PREC,
                         preferred_element_type=jnp.float32) + bf2_ref[...]


def _pool(h, seg, wf1, bf1, wf2, bf2):
    return pl.pallas_call(
        _pool_body,
        out_shape=jax.ShapeDtypeStruct((N_GRAPHS, 1), jnp.float32),
        scratch_shapes=[pltpu.VMEM((N_GRAPHS, D), jnp.float32)],
    )(h, seg, wf1, bf1, wf2, bf2)


def kernel(x, edge_index, batch,
           W1a, b1a, W1b, b1b, W2a, b2a, W2b, b2b, Wf1, bf1, Wf2, bf2):
    src = edge_index[0].astype(jnp.int32)
    dst = edge_index[1].astype(jnp.int32)
    pad = _E_PAD - N_EDGES
    srcs = jnp.concatenate([src, jnp.zeros((pad,), jnp.int32)])
    srcs = srcs.reshape(_NW, _CH, _CHUNK)
    dsts = jnp.concatenate([dst, jnp.full((pad,), N_NODES, jnp.int32)])
    dsts = dsts.reshape(_NW, _CH, _CHUNK)
    zeros = jnp.zeros((_RPT, D), jnp.float32)

    p = _sc_agg(x, srcs, dsts, zeros)
    h1 = _mlp(x, p, W1a, b1a.reshape(1, D), W1b, b1b.reshape(1, D))
    q = _sc_agg(h1, srcs, dsts, zeros)
    h2 = _mlp(h1, q, W2a, b2a.reshape(1, D), W2b, b2b.reshape(1, D))
    return _pool(h2, batch.astype(jnp.int32).reshape(N_NODES, 1),
                 Wf1, bf1.reshape(1, D), Wf2, bf2.reshape(1, 1))
